# baseline (speedup 1.0000x reference)
import functools

import jax
import jax.numpy as jnp
from jax import lax
from jax.experimental import pallas as pl
from jax.experimental.pallas import tpu as pltpu
from jax.experimental.pallas import tpu_sc as plsc

N = 10000
E = 320000
D = 128
DE = 16
H = 128
GS = 32
L = 3

NC = 2
NS = 16
NW = NC * NS
CH = 128
KCH = (-(-E // (NW * CH)) + 7) // 8 * 8
KCH0 = 48
KCH1 = 2 * KCH - KCH0
EPAD = NW * KCH * CH
RPT = (-(-(N + 1) // NS) + 7) // 8 * 8
NPAD = NS * RPT
IB = 16

def _mesh():
  return plsc.VectorSubcoreMesh(
      core_axis_name="c", subcore_axis_name="s", num_cores=NC, num_subcores=NS)


def _zero_rows(ref, nrows, width):
  z = jnp.zeros((16,), jnp.float32)

  @pl.loop(0, nrows)
  def _(i):
    for k in range(width // 16):
      ref[i, pl.ds(k * 16, 16)] = z


def _zero_stripe(zsrc, acc, row0):
  nfull, rem = divmod(RPT, CH)
  for t in range(nfull):
    pltpu.sync_copy(zsrc.at[pl.ds(0, CH)], acc.at[pl.ds(row0 + t * CH, CH)])
  if rem:
    pltpu.sync_copy(zsrc.at[pl.ds(0, rem)], acc.at[pl.ds(row0 + nfull * CH, rem)])


def _write_stripe(acc, row0, out_hbm, out0):
  nfull, rem = divmod(RPT, CH)
  for t in range(nfull):
    pltpu.sync_copy(acc.at[pl.ds(row0 + t * CH, CH)],
                    out_hbm.at[pl.ds(out0 + t * CH, CH)])
  if rem:
    pltpu.sync_copy(acc.at[pl.ds(row0 + nfull * CH, rem)],
                    out_hbm.at[pl.ds(out0 + nfull * CH, rem)])


def _segsum_body(h_hbm, src_hbm, dst_hbm, out_hbm, sidx, didx, rows_a, rows_b,
                 acc, sem_a, sem_b):
  c = lax.axis_index("c")
  s = lax.axis_index("s")
  base = jnp.where(c == 0, s * KCH0, NS * KCH0 + s * KCH1)
  nsup = jnp.where(c == 0, KCH0 // IB, KCH1 // IB)
  row0 = s * RPT

  _zero_rows(rows_a, CH, H)
  _zero_stripe(rows_a, acc, row0)
  plsc.subcore_barrier()

  bufs = [(rows_a, sem_a), (rows_b, sem_b)]

  @pl.loop(0, nsup)
  def _(g):
    pltpu.sync_copy(src_hbm.at[pl.ds(base + g * IB, IB)], sidx)
    pltpu.sync_copy(dst_hbm.at[pl.ds(base + g * IB, IB)], didx)
    d = pltpu.async_copy(h_hbm.at[sidx.at[0]], bufs[0][0], bufs[0][1])
    for jj in range(IB):
      cur, _ = bufs[jj % 2]
      if jj + 1 < IB:
        nbuf, nsem = bufs[(jj + 1) % 2]
        d_next = pltpu.async_copy(h_hbm.at[sidx.at[jj + 1]], nbuf, nsem)
      d.wait()
      pltpu.sync_copy(cur, acc.at[didx.at[jj]], add=True)
      if jj + 1 < IB:
        d = d_next

  plsc.subcore_barrier()
  _write_stripe(acc, row0, out_hbm, c * NPAD + row0)


def _sc_segsum(h, srcp, dstp):
  fn = pl.kernel(
      _segsum_body,
      out_type=[jax.ShapeDtypeStruct((NC * NPAD, H), jnp.float32)],
      mesh=_mesh(),
      scratch_types=[
          pltpu.VMEM((IB, CH), jnp.int32),
          pltpu.VMEM((IB, CH), jnp.int32),
          pltpu.VMEM((CH, H), jnp.float32),
          pltpu.VMEM((CH, H), jnp.float32),
          pltpu.VMEM_SHARED((NPAD, H), jnp.float32),
          pltpu.SemaphoreType.DMA,
          pltpu.SemaphoreType.DMA,
      ],
  )
  return fn(h, srcp, dstp)[0]


def _easum_body(ea_hbm, dst_hbm, out_hbm, didx, ear16, rows, acc):
  c = lax.axis_index("c")
  s = lax.axis_index("s")
  w = c * NS + s
  base = w * KCH
  row0 = s * RPT

  _zero_rows(rows, CH, H)
  _zero_stripe(rows, acc, row0)
  plsc.subcore_barrier()

  @pl.loop(0, KCH // IB)
  def _(g):
    pltpu.sync_copy(dst_hbm.at[pl.ds(base + g * IB, IB)], didx)
    for jj in range(IB):
      pltpu.sync_copy(ea_hbm.at[pl.ds((base + g * IB + jj) * CH, CH)], ear16)

      @pl.loop(0, CH)
      def _(i):
        rows[i, pl.ds(0, DE)] = ear16[i, pl.ds(0, DE)]

      pltpu.sync_copy(rows, acc.at[didx.at[jj]], add=True)

  plsc.subcore_barrier()
  _write_stripe(acc, row0, out_hbm, c * NPAD + row0)


def _sc_easum(eap, dstp):
  fn = pl.kernel(
      _easum_body,
      out_type=[jax.ShapeDtypeStruct((NC * NPAD, H), jnp.float32)],
      mesh=_mesh(),
      scratch_types=[
          pltpu.VMEM((IB, CH), jnp.int32),
          pltpu.VMEM((CH, DE), jnp.float32),
          pltpu.VMEM((CH, H), jnp.float32),
          pltpu.VMEM_SHARED((NPAD, H), jnp.float32),
      ],
  )
  return fn(eap, dstp)[0]


BLK = 2000


def _full(shape):
  return pl.BlockSpec(shape, lambda i: (0,) * len(shape))


def _rows(width):
  return pl.BlockSpec((BLK, width), lambda i: (i, 0))


def _tc_input_kernel(x_ref, wt_ref, b_ref, hi_ref, h0_ref):
  hi = jnp.dot(x_ref[...], wt_ref[...], preferred_element_type=jnp.float32)
  hi = hi + b_ref[...]
  hi_ref[...] = hi
  h0_ref[...] = jnp.maximum(hi, 0.0)


def _tc_input(x, WiT, bi):
  return pl.pallas_call(
      _tc_input_kernel,
      grid=(N // BLK,),
      in_specs=[_rows(D), _full((D, H)), _full((1, H))],
      out_specs=[_rows(H), _rows(H)],
      out_shape=[jax.ShapeDtypeStruct((N, H), jnp.float32)] * 2,
  )(x, WiT, bi)


def _tc_layer1_kernel(hi_ref, s0_ref, s1_ref, e0_ref, e1_ref,
                      wm1t_ref, wm2t_ref, bm_ref, hb_ref, h1_ref):
  ea = e0_ref[...] + e1_ref[...]
  hb = (hi_ref[...] + bm_ref[...]
        + jnp.dot(ea, wm2t_ref[...], preferred_element_type=jnp.float32))
  hb_ref[...] = hb
  sm = s0_ref[...] + s1_ref[...]
  h1_ref[...] = jnp.maximum(
      hb + jnp.dot(sm, wm1t_ref[...], preferred_element_type=jnp.float32), 0.0)


def _tc_layer1(hi, s0, s1, e0, e1, Wm1T, Wm2T, bm):
  return pl.pallas_call(
      _tc_layer1_kernel,
      grid=(N // BLK,),
      in_specs=[_rows(H), _rows(H), _rows(H), _rows(H), _rows(H),
                _full((H, H)), _full((H, H)), _full((1, H))],
      out_specs=[_rows(H), _rows(H)],
      out_shape=[jax.ShapeDtypeStruct((N, H), jnp.float32)] * 2,
  )(hi, s0, s1, e0, e1, Wm1T, Wm2T, bm)


def _tc_layer_kernel(hb_ref, s0_ref, s1_ref, wm1t_ref, h_ref):
  sm = s0_ref[...] + s1_ref[...]
  h_ref[...] = jnp.maximum(
      hb_ref[...]
      + jnp.dot(sm, wm1t_ref[...], preferred_element_type=jnp.float32), 0.0)


def _tc_layer(hb, s0, s1, Wm1T):
  return pl.pallas_call(
      _tc_layer_kernel,
      grid=(N // BLK,),
      in_specs=[_rows(H), _rows(H), _rows(H), _full((H, H))],
      out_specs=_rows(H),
      out_shape=jax.ShapeDtypeStruct((N, H), jnp.float32),
  )(hb, s0, s1, Wm1T)


def _tc_head_kernel(x_ref, m0_ref, m1_ref, waxt_ref, wamt_ref, ba_ref,
                    g_ref, wr1ht_ref, wr1gt_ref, br1_ref, wr2t_ref, br2_ref,
                    wr3t_ref, br3_ref, out_ref, acc_ref):
  i = pl.program_id(0)

  @pl.when(i == 0)
  def _():
    acc_ref[...] = jnp.zeros_like(acc_ref)

  m = m0_ref[...] + m1_ref[...]
  hv = jnp.maximum(
      jnp.dot(x_ref[...], waxt_ref[...], preferred_element_type=jnp.float32)
      + jnp.dot(m, wamt_ref[...], preferred_element_type=jnp.float32)
      + ba_ref[...], 0.0)
  acc_ref[...] += jnp.sum(hv, axis=0, keepdims=True)

  @pl.when(i == pl.num_programs(0) - 1)
  def _():
    mean = acc_ref[...] * (1.0 / N)
    r1 = jnp.maximum(
        jnp.dot(mean, wr1ht_ref[...], preferred_element_type=jnp.float32)
        + jnp.dot(g_ref[...], wr1gt_ref[...], preferred_element_type=jnp.float32)
        + br1_ref[...], 0.0)
    r2 = jnp.maximum(
        jnp.dot(r1, wr2t_ref[...], preferred_element_type=jnp.float32)
        + br2_ref[...], 0.0)
    out_ref[...] = (jnp.dot(r2, wr3t_ref[...], preferred_element_type=jnp.float32)
                    + br3_ref[...])


def _tc_head(x, m0, m1, WaxT, WamT, ba, g, Wr1hT, Wr1gT, br1, Wr2T, br2,
             Wr3T, br3, R1, R2, T):
  return pl.pallas_call(
      _tc_head_kernel,
      grid=(N // BLK,),
      in_specs=[_rows(D), _rows(H), _rows(H),
                _full((D, H)), _full((H, H)), _full((1, H)),
                _full((1, GS)), _full((H, R1)), _full((GS, R1)), _full((1, R1)),
                _full((R1, R2)), _full((1, R2)), _full((R2, T)), _full((1, T))],
      out_specs=pl.BlockSpec((1, T), lambda i: (0, 0)),
      out_shape=jax.ShapeDtypeStruct((1, T), jnp.float32),
      scratch_shapes=[pltpu.VMEM((1, H), jnp.float32)],
  )(x, m0, m1, WaxT, WamT, ba, g, Wr1hT, Wr1gT, br1, Wr2T, br2, Wr3T, br3)


def kernel(x, edge_index, edge_attr, globals_feat, Wi, bi, Wm, bm,
           Wa, ba, Wr1, br1, Wr2, br2, Wr3, br3):
  R1 = Wr1.shape[0]
  R2 = Wr2.shape[0]
  T = Wr3.shape[0]

  pad = EPAD - E
  src = jnp.concatenate([edge_index[0], jnp.zeros((pad,), jnp.int32)])
  dst = jnp.concatenate([edge_index[1], jnp.full((pad,), N, jnp.int32)])
  srcp = src.reshape(NW * KCH, CH)
  dstp = dst.reshape(NW * KCH, CH)
  eap = jnp.concatenate([edge_attr, jnp.zeros((pad, DE), jnp.float32)])

  WiT = Wi.T
  Wm1T = Wm[:, :H].T
  Wm2Tp = jnp.zeros((H, H), jnp.float32).at[:DE].set(Wm[:, H:].T)
  WaxT = Wa[:, :D].T
  WamT = Wa[:, D:].T
  Wr1hT = Wr1[:, :H].T
  Wr1gT = Wr1[:, H:].T
  Wr2T = Wr2.T
  Wr3T = Wr3.T

  h_input, h = _tc_input(x, WiT, bi.reshape(1, H))
  epart = _sc_easum(eap, dstp)

  hb = None
  for layer in range(L):
    spart = _sc_segsum(h, srcp, dstp)
    if layer == 0:
      hb, h = _tc_layer1(h_input,
                         spart[:N], spart[NPAD:NPAD + N],
                         epart[:N], epart[NPAD:NPAD + N],
                         Wm1T, Wm2Tp, bm.reshape(1, H))
    else:
      h = _tc_layer(hb, spart[:N], spart[NPAD:NPAD + N], Wm1T)

  mpart = _sc_segsum(h, srcp, dstp)
  out = _tc_head(x, mpart[:N], mpart[NPAD:NPAD + N],
                 WaxT, WamT, ba.reshape(1, H), globals_feat,
                 Wr1hT, Wr1gT, br1.reshape(1, R1),
                 Wr2T, br2.reshape(1, R2), Wr3T, br3.reshape(1, T),
                 R1, R2, T)
  return out


# speedup vs baseline: 1.0669x; 1.0669x over previous
import functools

import jax
import jax.numpy as jnp
from jax import lax
from jax.experimental import pallas as pl
from jax.experimental.pallas import tpu as pltpu
from jax.experimental.pallas import tpu_sc as plsc

N = 10000
E = 320000
D = 128
DE = 16
H = 128
GS = 32
L = 3

NC = 2
NS = 16
NW = NC * NS
CH = 128
KCH = (-(-E // (NW * CH)) + 7) // 8 * 8
KCH0 = 112
KCH1 = 2 * KCH - KCH0
EPAD = NW * KCH * CH
RPT = (-(-(N + 1) // NS) + 7) // 8 * 8
NPAD = NS * RPT
IB = 16

def _mesh():
  return plsc.VectorSubcoreMesh(
      core_axis_name="c", subcore_axis_name="s", num_cores=NC, num_subcores=NS)


def _zero_rows(ref, nrows, width):
  z = jnp.zeros((16,), jnp.float32)

  @pl.loop(0, nrows)
  def _(i):
    for k in range(width // 16):
      ref[i, pl.ds(k * 16, 16)] = z


def _zero_stripe(zsrc, acc, row0):
  nfull, rem = divmod(RPT, CH)
  for t in range(nfull):
    pltpu.sync_copy(zsrc.at[pl.ds(0, CH)], acc.at[pl.ds(row0 + t * CH, CH)])
  if rem:
    pltpu.sync_copy(zsrc.at[pl.ds(0, rem)], acc.at[pl.ds(row0 + nfull * CH, rem)])


def _write_stripe(acc, row0, out_hbm, out0):
  nfull, rem = divmod(RPT, CH)
  for t in range(nfull):
    pltpu.sync_copy(acc.at[pl.ds(row0 + t * CH, CH)],
                    out_hbm.at[pl.ds(out0 + t * CH, CH)])
  if rem:
    pltpu.sync_copy(acc.at[pl.ds(row0 + nfull * CH, rem)],
                    out_hbm.at[pl.ds(out0 + nfull * CH, rem)])


def _segsum_body(h_hbm, src_hbm, dst_hbm, out_hbm, sidx, didx, rows_a, rows_b,
                 acc, sem_a, sem_b):
  c = lax.axis_index("c")
  s = lax.axis_index("s")
  base = jnp.where(c == 0, s * KCH0, NS * KCH0 + s * KCH1)
  nsup = jnp.where(c == 0, KCH0 // IB, KCH1 // IB)
  row0 = s * RPT

  _zero_rows(rows_a, CH, H)
  _zero_stripe(rows_a, acc, row0)
  plsc.subcore_barrier()

  bufs = [(rows_a, sem_a), (rows_b, sem_b)]

  @pl.loop(0, nsup)
  def _(g):
    pltpu.sync_copy(src_hbm.at[pl.ds(base + g * IB, IB)], sidx)
    pltpu.sync_copy(dst_hbm.at[pl.ds(base + g * IB, IB)], didx)
    d = pltpu.async_copy(h_hbm.at[sidx.at[0]], bufs[0][0], bufs[0][1])
    for jj in range(IB):
      cur, _ = bufs[jj % 2]
      if jj + 1 < IB:
        nbuf, nsem = bufs[(jj + 1) % 2]
        d_next = pltpu.async_copy(h_hbm.at[sidx.at[jj + 1]], nbuf, nsem)
      d.wait()
      pltpu.sync_copy(cur, acc.at[didx.at[jj]], add=True)
      if jj + 1 < IB:
        d = d_next

  plsc.subcore_barrier()
  _write_stripe(acc, row0, out_hbm, c * NPAD + row0)


def _sc_segsum(h, srcp, dstp):
  fn = pl.kernel(
      _segsum_body,
      out_type=[jax.ShapeDtypeStruct((NC * NPAD, H), jnp.float32)],
      mesh=_mesh(),
      scratch_types=[
          pltpu.VMEM((IB, CH), jnp.int32),
          pltpu.VMEM((IB, CH), jnp.int32),
          pltpu.VMEM((CH, H), jnp.float32),
          pltpu.VMEM((CH, H), jnp.float32),
          pltpu.VMEM_SHARED((NPAD, H), jnp.float32),
          pltpu.SemaphoreType.DMA,
          pltpu.SemaphoreType.DMA,
      ],
  )
  return fn(h, srcp, dstp)[0]


def _easum_body(ea_hbm, dst_hbm, out_hbm, didx, ear16, rows, acc):
  c = lax.axis_index("c")
  s = lax.axis_index("s")
  w = c * NS + s
  base = w * KCH
  row0 = s * RPT

  _zero_rows(rows, CH, H)
  _zero_stripe(rows, acc, row0)
  plsc.subcore_barrier()

  @pl.loop(0, KCH // IB)
  def _(g):
    pltpu.sync_copy(dst_hbm.at[pl.ds(base + g * IB, IB)], didx)
    for jj in range(IB):
      pltpu.sync_copy(ea_hbm.at[pl.ds((base + g * IB + jj) * CH, CH)], ear16)

      @pl.loop(0, CH)
      def _(i):
        rows[i, pl.ds(0, DE)] = ear16[i, pl.ds(0, DE)]

      pltpu.sync_copy(rows, acc.at[didx.at[jj]], add=True)

  plsc.subcore_barrier()
  _write_stripe(acc, row0, out_hbm, c * NPAD + row0)


def _sc_easum(eap, dstp):
  fn = pl.kernel(
      _easum_body,
      out_type=[jax.ShapeDtypeStruct((NC * NPAD, H), jnp.float32)],
      mesh=_mesh(),
      scratch_types=[
          pltpu.VMEM((IB, CH), jnp.int32),
          pltpu.VMEM((CH, DE), jnp.float32),
          pltpu.VMEM((CH, H), jnp.float32),
          pltpu.VMEM_SHARED((NPAD, H), jnp.float32),
      ],
  )
  return fn(eap, dstp)[0]


BLK = 2000


def _full(shape):
  return pl.BlockSpec(shape, lambda i: (0,) * len(shape))


def _rows(width):
  return pl.BlockSpec((BLK, width), lambda i: (i, 0))


def _tc_input_kernel(x_ref, wt_ref, b_ref, hi_ref, h0_ref):
  hi = jnp.dot(x_ref[...], wt_ref[...], preferred_element_type=jnp.float32)
  hi = hi + b_ref[...]
  hi_ref[...] = hi
  h0_ref[...] = jnp.maximum(hi, 0.0)


def _tc_input(x, WiT, bi):
  return pl.pallas_call(
      _tc_input_kernel,
      grid=(N // BLK,),
      in_specs=[_rows(D), _full((D, H)), _full((1, H))],
      out_specs=[_rows(H), _rows(H)],
      out_shape=[jax.ShapeDtypeStruct((N, H), jnp.float32)] * 2,
  )(x, WiT, bi)


def _tc_layer1_kernel(hi_ref, s0_ref, s1_ref, e0_ref, e1_ref,
                      wm1t_ref, wm2t_ref, bm_ref, hb_ref, h1_ref):
  ea = e0_ref[...] + e1_ref[...]
  hb = (hi_ref[...] + bm_ref[...]
        + jnp.dot(ea, wm2t_ref[...], preferred_element_type=jnp.float32))
  hb_ref[...] = hb
  sm = s0_ref[...] + s1_ref[...]
  h1_ref[...] = jnp.maximum(
      hb + jnp.dot(sm, wm1t_ref[...], preferred_element_type=jnp.float32), 0.0)


def _tc_layer1(hi, s0, s1, e0, e1, Wm1T, Wm2T, bm):
  return pl.pallas_call(
      _tc_layer1_kernel,
      grid=(N // BLK,),
      in_specs=[_rows(H), _rows(H), _rows(H), _rows(H), _rows(H),
                _full((H, H)), _full((H, H)), _full((1, H))],
      out_specs=[_rows(H), _rows(H)],
      out_shape=[jax.ShapeDtypeStruct((N, H), jnp.float32)] * 2,
  )(hi, s0, s1, e0, e1, Wm1T, Wm2T, bm)


def _tc_layer_kernel(hb_ref, s0_ref, s1_ref, wm1t_ref, h_ref):
  sm = s0_ref[...] + s1_ref[...]
  h_ref[...] = jnp.maximum(
      hb_ref[...]
      + jnp.dot(sm, wm1t_ref[...], preferred_element_type=jnp.float32), 0.0)


def _tc_layer(hb, s0, s1, Wm1T):
  return pl.pallas_call(
      _tc_layer_kernel,
      grid=(N // BLK,),
      in_specs=[_rows(H), _rows(H), _rows(H), _full((H, H))],
      out_specs=_rows(H),
      out_shape=jax.ShapeDtypeStruct((N, H), jnp.float32),
  )(hb, s0, s1, Wm1T)


def _tc_head_kernel(x_ref, m0_ref, m1_ref, waxt_ref, wamt_ref, ba_ref,
                    g_ref, wr1ht_ref, wr1gt_ref, br1_ref, wr2t_ref, br2_ref,
                    wr3t_ref, br3_ref, out_ref, acc_ref):
  i = pl.program_id(0)

  @pl.when(i == 0)
  def _():
    acc_ref[...] = jnp.zeros_like(acc_ref)

  m = m0_ref[...] + m1_ref[...]
  hv = jnp.maximum(
      jnp.dot(x_ref[...], waxt_ref[...], preferred_element_type=jnp.float32)
      + jnp.dot(m, wamt_ref[...], preferred_element_type=jnp.float32)
      + ba_ref[...], 0.0)
  acc_ref[...] += jnp.sum(hv, axis=0, keepdims=True)

  @pl.when(i == pl.num_programs(0) - 1)
  def _():
    mean = acc_ref[...] * (1.0 / N)
    r1 = jnp.maximum(
        jnp.dot(mean, wr1ht_ref[...], preferred_element_type=jnp.float32)
        + jnp.dot(g_ref[...], wr1gt_ref[...], preferred_element_type=jnp.float32)
        + br1_ref[...], 0.0)
    r2 = jnp.maximum(
        jnp.dot(r1, wr2t_ref[...], preferred_element_type=jnp.float32)
        + br2_ref[...], 0.0)
    out_ref[...] = (jnp.dot(r2, wr3t_ref[...], preferred_element_type=jnp.float32)
                    + br3_ref[...])


def _tc_head(x, m0, m1, WaxT, WamT, ba, g, Wr1hT, Wr1gT, br1, Wr2T, br2,
             Wr3T, br3, R1, R2, T):
  return pl.pallas_call(
      _tc_head_kernel,
      grid=(N // BLK,),
      in_specs=[_rows(D), _rows(H), _rows(H),
                _full((D, H)), _full((H, H)), _full((1, H)),
                _full((1, GS)), _full((H, R1)), _full((GS, R1)), _full((1, R1)),
                _full((R1, R2)), _full((1, R2)), _full((R2, T)), _full((1, T))],
      out_specs=pl.BlockSpec((1, T), lambda i: (0, 0)),
      out_shape=jax.ShapeDtypeStruct((1, T), jnp.float32),
      scratch_shapes=[pltpu.VMEM((1, H), jnp.float32)],
  )(x, m0, m1, WaxT, WamT, ba, g, Wr1hT, Wr1gT, br1, Wr2T, br2, Wr3T, br3)


def kernel(x, edge_index, edge_attr, globals_feat, Wi, bi, Wm, bm,
           Wa, ba, Wr1, br1, Wr2, br2, Wr3, br3):
  R1 = Wr1.shape[0]
  R2 = Wr2.shape[0]
  T = Wr3.shape[0]

  pad = EPAD - E
  src = jnp.concatenate([edge_index[0], jnp.zeros((pad,), jnp.int32)])
  dst = jnp.concatenate([edge_index[1], jnp.full((pad,), N, jnp.int32)])
  srcp = src.reshape(NW * KCH, CH)
  dstp = dst.reshape(NW * KCH, CH)
  eap = jnp.concatenate([edge_attr, jnp.zeros((pad, DE), jnp.float32)])

  WiT = Wi.T
  Wm1T = Wm[:, :H].T
  Wm2Tp = jnp.zeros((H, H), jnp.float32).at[:DE].set(Wm[:, H:].T)
  WaxT = Wa[:, :D].T
  WamT = Wa[:, D:].T
  Wr1hT = Wr1[:, :H].T
  Wr1gT = Wr1[:, H:].T
  Wr2T = Wr2.T
  Wr3T = Wr3.T

  h_input, h = _tc_input(x, WiT, bi.reshape(1, H))
  epart = _sc_easum(eap, dstp)

  hb = None
  for layer in range(L):
    spart = _sc_segsum(h, srcp, dstp)
    if layer == 0:
      hb, h = _tc_layer1(h_input,
                         spart[:N], spart[NPAD:NPAD + N],
                         epart[:N], epart[NPAD:NPAD + N],
                         Wm1T, Wm2Tp, bm.reshape(1, H))
    else:
      h = _tc_layer(hb, spart[:N], spart[NPAD:NPAD + N], Wm1T)

  mpart = _sc_segsum(h, srcp, dstp)
  out = _tc_head(x, mpart[:N], mpart[NPAD:NPAD + N],
                 WaxT, WamT, ba.reshape(1, H), globals_feat,
                 Wr1hT, Wr1gT, br1.reshape(1, R1),
                 Wr2T, br2.reshape(1, R2), Wr3T, br3.reshape(1, T),
                 R1, R2, T)
  return out
